# baseline (speedup 1.0000x reference)
import jax
import jax.numpy as jnp
from jax import lax
from jax.experimental import pallas as pl
from jax.experimental.pallas import tpu as pltpu
from jax.experimental.pallas import tpu_sc as plsc

N = 10000
E = 320000
F_IN = 128
HID = 128
OUT = 128
G = 128

NS = 16
L = 16

NPAD = 10240
EW = E // NS
NT = NPAD // NS
CH = 2048
CSL = NPAD * G // NS
ZB = 2048

CHUNKS = [(k * CH, CH, CH // 128) for k in range(9)] + [(9 * CH, 1568, 13)]


def _rsqrt16(x):
    i = plsc.bitcast(x, jnp.int32)
    y = plsc.bitcast(jnp.int32(0x5F3759DF) - (i >> 1), jnp.float32)
    for _ in range(3):
        y = y * (1.5 - 0.5 * x * y * y)
    return y


def _sc_body(srce_hbm, dste_hbm, batch_hbm, c_out, cnt_out,
             batch_v, dinv_v, src_c, dst_c, acc_s, zbuf_v, idx_c, val_c,
             C_sh, deg_sh, dinv_sh, cnt_sh):
    sid = lax.axis_index("s")

    zero16f = jnp.zeros((L,), jnp.float32)
    zero16i = jnp.zeros((L,), jnp.int32)
    one16f = jnp.ones((L,), jnp.float32)
    ebase = sid * EW
    sb = sid * NT

    pltpu.sync_copy(batch_hbm, batch_v.at[pl.ds(0, N)])
    for q in range((NPAD - N) // L):
        batch_v[pl.ds(N + q * L, L)] = zero16i

    def zz(i, c):
        zbuf_v[pl.ds(i * L, L)] = zero16f
        return c
    lax.fori_loop(0, ZB // L, zz, 0)
    for q in range(CSL // ZB):
        pltpu.sync_copy(zbuf_v, C_sh.at[pl.ds(sid * CSL + q * ZB, ZB)])
    pltpu.sync_copy(zbuf_v.at[pl.ds(0, NT)], deg_sh.at[pl.ds(sb, NT)])

    @pl.when(sid == 0)
    def _():
        pltpu.sync_copy(zbuf_v.at[pl.ds(0, G)], cnt_sh)

    plsc.subcore_barrier()

    def fill_ones(i, c):
        val_c[i // 8, pl.ds((i % 8) * L, L)] = one16f
        return c
    lax.fori_loop(0, CH // L, fill_ones, 0)

    for (cb, cn, rows) in CHUNKS:
        pltpu.sync_copy(dste_hbm.at[pl.ds(ebase + cb, cn)],
                        dst_c.at[pl.ds(0, cn)])

        def degidx(i, c):
            idx_c[i // 8, pl.ds((i % 8) * L, L)] = dst_c[pl.ds(i * L, L)]
            return c
        lax.fori_loop(0, cn // L, degidx, 0)
        if cn % 128:
            for q in range((128 - cn % 128) // L):
                k = cn + q * L
                idx_c[k // 128, pl.ds(k % 128, L)] = zero16i
                val_c[k // 128, pl.ds(k % 128, L)] = zero16f
        for j in range(rows):
            pltpu.sync_copy(val_c.at[j], deg_sh.at[idx_c.at[j]], add=True)
        if cn % 128:
            for q in range((128 - cn % 128) // L):
                k = cn + q * L
                val_c[k // 128, pl.ds(k % 128, L)] = one16f

    plsc.subcore_barrier()

    pltpu.sync_copy(deg_sh.at[pl.ds(sb, NT)], acc_s)

    def dinv_calc(q, c):
        acc_s[pl.ds(q * L, L)] = _rsqrt16(acc_s[pl.ds(q * L, L)] + 1.0)
        return c
    lax.fori_loop(0, NT // L, dinv_calc, 0)
    pltpu.sync_copy(acc_s, dinv_sh.at[pl.ds(sb, NT)])

    plsc.subcore_barrier()
    pltpu.sync_copy(dinv_sh, dinv_v)

    for (cb, cn, rows) in CHUNKS:
        pltpu.sync_copy(srce_hbm.at[pl.ds(ebase + cb, cn)],
                        src_c.at[pl.ds(0, cn)])
        pltpu.sync_copy(dste_hbm.at[pl.ds(ebase + cb, cn)],
                        dst_c.at[pl.ds(0, cn)])

        def coeff(i, c):
            s16 = src_c[pl.ds(i * L, L)]
            d16 = dst_c[pl.ds(i * L, L)]
            a = plsc.load_gather(dinv_v, [s16])
            b = plsc.load_gather(dinv_v, [d16])
            g16 = plsc.load_gather(batch_v, [d16])
            idx_c[i // 8, pl.ds((i % 8) * L, L)] = s16 * G + g16
            val_c[i // 8, pl.ds((i % 8) * L, L)] = a * b
            return c
        lax.fori_loop(0, cn // L, coeff, 0)
        if cn % 128:
            for q in range((128 - cn % 128) // L):
                k = cn + q * L
                idx_c[k // 128, pl.ds(k % 128, L)] = zero16i
                val_c[k // 128, pl.ds(k % 128, L)] = zero16f
        for j in range(rows):
            pltpu.sync_copy(val_c.at[j], C_sh.at[idx_c.at[j]], add=True)

    for q in range(NT // L):
        k = q * L
        nvec = sb + k + lax.broadcasted_iota(jnp.int32, (L,), 0)
        dv = dinv_v[pl.ds(sb + k, L)]
        g16 = batch_v[pl.ds(sb + k, L)]
        valid = nvec < N
        idx_c[k // 128, pl.ds(k % 128, L)] = jnp.where(valid, nvec * G + g16, 0)
        val_c[k // 128, pl.ds(k % 128, L)] = jnp.where(valid, dv * dv, 0.0)
    for j in range(NT // 128):
        pltpu.sync_copy(val_c.at[j], C_sh.at[idx_c.at[j]], add=True)

    for q in range(NT // L):
        k = q * L
        nvec = sb + k + lax.broadcasted_iota(jnp.int32, (L,), 0)
        g16 = batch_v[pl.ds(sb + k, L)]
        valid = nvec < N
        idx_c[k // 128, pl.ds(k % 128, L)] = jnp.where(valid, g16, 0)
        val_c[k // 128, pl.ds(k % 128, L)] = jnp.where(valid, 1.0, 0.0)
    for j in range(NT // 128):
        pltpu.sync_copy(val_c.at[j], cnt_sh.at[idx_c.at[j]], add=True)

    plsc.subcore_barrier()

    pltpu.sync_copy(C_sh.at[pl.ds(sid * CSL, CSL)],
                    c_out.at[pl.ds(sid * CSL, CSL)])

    @pl.when(sid == 0)
    def _():
        pltpu.sync_copy(cnt_sh, cnt_out)


@jax.jit
def _sc_coeffs(srce, dste, batch):
    mesh = plsc.VectorSubcoreMesh(core_axis_name="c", subcore_axis_name="s",
                                  num_cores=1, num_subcores=NS)
    f = pl.kernel(
        _sc_body,
        out_type=(jax.ShapeDtypeStruct((NPAD * G,), jnp.float32),
                  jax.ShapeDtypeStruct((G,), jnp.float32)),
        mesh=mesh,
        compiler_params=pltpu.CompilerParams(needs_layout_passes=False),
        scratch_types=[
            pltpu.VMEM((NPAD,), jnp.int32),
            pltpu.VMEM((NPAD,), jnp.float32),
            pltpu.VMEM((CH,), jnp.int32),
            pltpu.VMEM((CH,), jnp.int32),
            pltpu.VMEM((NT,), jnp.float32),
            pltpu.VMEM((ZB,), jnp.float32),
            pltpu.VMEM((CH // 128, 128), jnp.int32),
            pltpu.VMEM((CH // 128, 128), jnp.float32),
            pltpu.VMEM_SHARED((NPAD * G,), jnp.float32),
            pltpu.VMEM_SHARED((NPAD,), jnp.float32),
            pltpu.VMEM_SHARED((NPAD,), jnp.float32),
            pltpu.VMEM_SHARED((G,), jnp.float32),
        ],
    )
    return f(srce, dste, batch)


def _tc_body(c_ref, cnt_ref, x_ref, w1_ref, b1_ref, wlin_ref, blin_ref, out_ref):
    cs = c_ref[:N, :]
    a = lax.dot_general(cs, x_ref[...], (((0,), (0,)), ((), ())),
                        preferred_element_type=jnp.float32,
                        precision=lax.Precision.HIGHEST)
    cnt = cnt_ref[...]
    h = jnp.dot(a, w1_ref[...], preferred_element_type=jnp.float32,
                precision=lax.Precision.HIGHEST)
    h = h + cnt[:, None] * b1_ref[...][None, :]
    pooled = h / jnp.maximum(cnt, 1.0)[:, None]
    out_ref[...] = (jnp.dot(pooled, wlin_ref[...],
                            preferred_element_type=jnp.float32,
                            precision=lax.Precision.HIGHEST)
                    + blin_ref[...][None, :])


@jax.jit
def _tc_dense(c_flat, cnt, x, W1, b1, Wlin, blin):
    c2 = c_flat.reshape(NPAD, G)
    return pl.pallas_call(
        _tc_body,
        out_shape=jax.ShapeDtypeStruct((G, OUT), jnp.float32),
    )(c2, cnt, x, W1, b1, Wlin, blin)


def kernel(x, edge_index, batch, W1, b1, Wlin, blin):
    c_flat, cnt = _sc_coeffs(edge_index[0], edge_index[1], batch)
    return _tc_dense(c_flat, cnt, x, W1, b1, Wlin, blin)


# speedup vs baseline: 1.3816x; 1.3816x over previous
import jax
import jax.numpy as jnp
from jax import lax
from jax.experimental import pallas as pl
from jax.experimental.pallas import tpu as pltpu
from jax.experimental.pallas import tpu_sc as plsc

N = 10000
E = 320000
F_IN = 128
HID = 128
OUT = 128
G = 128

NS = 16
L = 16

NPAD = 10240
NT = NPAD // NS
RPT = 160
EROWS = NS * RPT
CSL = NPAD * G // NS
ZB = 2048

CHUNKS = [(q * 16, 16) for q in range(10)]


def _rsqrt16(x):
    i = plsc.bitcast(x, jnp.int32)
    y = plsc.bitcast(jnp.int32(0x5F3759DF) - (i >> 1), jnp.float32)
    for _ in range(3):
        y = y * (1.5 - 0.5 * x * y * y)
    return y


def _sc_body(srce_hbm, dste_hbm, batch_hbm, c_out, cnt_out,
             batch_v, dinv_v, src_c, dst_c, acc_s, zbuf_v, idx_c, val_c,
             C_sh, deg_sh, dinv_sh, cnt_sh, sem_stage, sem_scat, sem_zero):
    sid = lax.axis_index("s")

    zero16f = jnp.zeros((L,), jnp.float32)
    zero16i = jnp.zeros((L,), jnp.int32)
    one16f = jnp.ones((L,), jnp.float32)
    iota16 = lax.broadcasted_iota(jnp.int32, (L,), 0)
    rbase = sid * RPT
    sb = sid * NT

    stage_batch = pltpu.async_copy(batch_hbm, batch_v.at[pl.ds(0, N)],
                                   sem_stage)

    def zz(i, c):
        zbuf_v[pl.ds(i * L, L)] = zero16f
        return c
    lax.fori_loop(0, ZB // L, zz, 0)
    zdescs = [pltpu.async_copy(zbuf_v,
                               C_sh.at[pl.ds(sid * CSL + q * ZB, ZB)],
                               sem_zero)
              for q in range(CSL // ZB)]
    zdescs.append(pltpu.async_copy(zbuf_v.at[pl.ds(0, NT)],
                                   deg_sh.at[pl.ds(sb, NT)], sem_zero))

    @pl.when(sid == 0)
    def _():
        pltpu.async_copy(zbuf_v.at[pl.ds(0, G)], cnt_sh, sem_zero).wait()

    stage_batch.wait()
    for q in range((NPAD - N) // L):
        batch_v[pl.ds(N + q * L, L)] = zero16i
    for d in zdescs:
        d.wait()

    for p in range(2):
        def fill1(i, c, p=p):
            val_c[p, i // 8, pl.ds((i % 8) * L, L)] = one16f
            return c
        lax.fori_loop(0, (16 * 128) // L, fill1, 0)

    plsc.subcore_barrier()

    stage_d = {0: pltpu.async_copy(
        dste_hbm.at[pl.ds(rbase + CHUNKS[0][0], CHUNKS[0][1])],
        idx_c.at[0, pl.ds(0, CHUNKS[0][1])], sem_stage)}
    scat_d = {}
    for k, (off, rk) in enumerate(CHUNKS):
        p = k % 2
        stage_d[k].wait()
        scat_d[k] = [pltpu.async_copy(val_c.at[p, j],
                                      deg_sh.at[idx_c.at[p, j]],
                                      sem_scat, add=True)
                     for j in range(rk)]
        if k - 1 in scat_d:
            for d in scat_d.pop(k - 1):
                d.wait()
        if k + 1 < len(CHUNKS):
            off2, rk2 = CHUNKS[k + 1]
            stage_d[k + 1] = pltpu.async_copy(
                dste_hbm.at[pl.ds(rbase + off2, rk2)],
                idx_c.at[1 - p, pl.ds(0, rk2)], sem_stage)
    for d in scat_d.pop(len(CHUNKS) - 1):
        d.wait()

    plsc.subcore_barrier()

    pltpu.sync_copy(deg_sh.at[pl.ds(sb, NT)], acc_s)

    def dinv_calc(q, c):
        acc_s[pl.ds(q * L, L)] = _rsqrt16(acc_s[pl.ds(q * L, L)] + 1.0)
        return c
    lax.fori_loop(0, NT // L, dinv_calc, 0)
    pltpu.sync_copy(acc_s, dinv_sh.at[pl.ds(sb, NT)])

    plsc.subcore_barrier()
    pltpu.sync_copy(dinv_sh, dinv_v)

    stage_d = {0: [pltpu.async_copy(
        srce_hbm.at[pl.ds(rbase + CHUNKS[0][0], CHUNKS[0][1])],
        src_c.at[0, pl.ds(0, CHUNKS[0][1])], sem_stage),
        pltpu.async_copy(
        dste_hbm.at[pl.ds(rbase + CHUNKS[0][0], CHUNKS[0][1])],
        dst_c.at[0, pl.ds(0, CHUNKS[0][1])], sem_stage)]}
    scat_d = {}
    for k, (off, rk) in enumerate(CHUNKS):
        p = k % 2
        for d in stage_d.pop(k):
            d.wait()
        if k - 2 in scat_d:
            for d in scat_d.pop(k - 2):
                d.wait()

        def coeff(i, c, off=off, p=p):
            r = i // 8
            cl = (i % 8) * L
            s16 = src_c[p, r, pl.ds(cl, L)]
            d16 = dst_c[p, r, pl.ds(cl, L)]
            a = plsc.load_gather(dinv_v, [s16])
            b = plsc.load_gather(dinv_v, [d16])
            g16 = plsc.load_gather(batch_v, [d16])
            idx_c[p, r, pl.ds(cl, L)] = s16 * G + g16
            val_c[p, r, pl.ds(cl, L)] = a * b
            return c
        lax.fori_loop(0, (rk * 128) // L, coeff, 0)
        scat_d[k] = [pltpu.async_copy(val_c.at[p, j],
                                      C_sh.at[idx_c.at[p, j]],
                                      sem_scat, add=True)
                     for j in range(rk)]
        if k + 1 < len(CHUNKS):
            off2, rk2 = CHUNKS[k + 1]
            stage_d[k + 1] = [pltpu.async_copy(
                srce_hbm.at[pl.ds(rbase + off2, rk2)],
                src_c.at[1 - p, pl.ds(0, rk2)], sem_stage),
                pltpu.async_copy(
                dste_hbm.at[pl.ds(rbase + off2, rk2)],
                dst_c.at[1 - p, pl.ds(0, rk2)], sem_stage)]
    for k in list(scat_d):
        for d in scat_d.pop(k):
            d.wait()

    for q in range(NT // L):
        k = q * L
        nvec = sb + k + iota16
        dv = dinv_v[pl.ds(sb + k, L)]
        g16 = batch_v[pl.ds(sb + k, L)]
        valid = nvec < N
        idx_c[0, k // 128, pl.ds(k % 128, L)] = jnp.where(
            valid, nvec * G + g16, 0)
        val_c[0, k // 128, pl.ds(k % 128, L)] = jnp.where(
            valid, dv * dv, 0.0)
    sl_d = [pltpu.async_copy(val_c.at[0, j], C_sh.at[idx_c.at[0, j]],
                             sem_scat, add=True)
            for j in range(NT // 128)]

    for q in range(NT // L):
        k = q * L
        nvec = sb + k + iota16
        g16 = batch_v[pl.ds(sb + k, L)]
        valid = nvec < N
        idx_c[1, k // 128, pl.ds(k % 128, L)] = jnp.where(valid, g16, 0)
        val_c[1, k // 128, pl.ds(k % 128, L)] = jnp.where(valid, 1.0, 0.0)
    sl_d += [pltpu.async_copy(val_c.at[1, j], cnt_sh.at[idx_c.at[1, j]],
                              sem_scat, add=True)
             for j in range(NT // 128)]
    for d in sl_d:
        d.wait()

    plsc.subcore_barrier()

    pltpu.sync_copy(C_sh.at[pl.ds(sid * CSL, CSL)],
                    c_out.at[pl.ds(sid * CSL, CSL)])

    @pl.when(sid == 0)
    def _():
        pltpu.sync_copy(cnt_sh, cnt_out)


@jax.jit
def _sc_coeffs(srce2d, dste2d, batch):
    mesh = plsc.VectorSubcoreMesh(core_axis_name="c", subcore_axis_name="s",
                                  num_cores=1, num_subcores=NS)
    f = pl.kernel(
        _sc_body,
        out_type=(jax.ShapeDtypeStruct((NPAD * G,), jnp.float32),
                  jax.ShapeDtypeStruct((G,), jnp.float32)),
        mesh=mesh,
        compiler_params=pltpu.CompilerParams(needs_layout_passes=False),
        scratch_types=[
            pltpu.VMEM((NPAD,), jnp.int32),
            pltpu.VMEM((NPAD,), jnp.float32),
            pltpu.VMEM((2, 16, 128), jnp.int32),
            pltpu.VMEM((2, 16, 128), jnp.int32),
            pltpu.VMEM((NT,), jnp.float32),
            pltpu.VMEM((ZB,), jnp.float32),
            pltpu.VMEM((2, 16, 128), jnp.int32),
            pltpu.VMEM((2, 16, 128), jnp.float32),
            pltpu.VMEM_SHARED((NPAD * G,), jnp.float32),
            pltpu.VMEM_SHARED((NPAD,), jnp.float32),
            pltpu.VMEM_SHARED((NPAD,), jnp.float32),
            pltpu.VMEM_SHARED((G,), jnp.float32),
            pltpu.SemaphoreType.DMA,
            pltpu.SemaphoreType.DMA,
            pltpu.SemaphoreType.DMA,
        ],
    )
    return f(srce2d, dste2d, batch)


def _tc_body(c_ref, cnt_ref, x_ref, w1_ref, b1_ref, wlin_ref, blin_ref, out_ref):
    cs = c_ref[:N, :]
    a = lax.dot_general(cs, x_ref[...], (((0,), (0,)), ((), ())),
                        preferred_element_type=jnp.float32,
                        precision=lax.Precision.HIGHEST)
    cnt = cnt_ref[...]
    h = jnp.dot(a, w1_ref[...], preferred_element_type=jnp.float32,
                precision=lax.Precision.HIGHEST)
    h = h + cnt[:, None] * b1_ref[...][None, :]
    pooled = h / jnp.maximum(cnt, 1.0)[:, None]
    out_ref[...] = (jnp.dot(pooled, wlin_ref[...],
                            preferred_element_type=jnp.float32,
                            precision=lax.Precision.HIGHEST)
                    + blin_ref[...][None, :])


@jax.jit
def _tc_dense(c_flat, cnt, x, W1, b1, Wlin, blin):
    c2 = c_flat.reshape(NPAD, G)
    return pl.pallas_call(
        _tc_body,
        out_shape=jax.ShapeDtypeStruct((G, OUT), jnp.float32),
    )(c2, cnt, x, W1, b1, Wlin, blin)


@jax.jit
def _prep(edge_index):
    pad = jnp.full((EROWS * 128 - E,), N, jnp.int32)
    srce2d = jnp.concatenate([edge_index[0], pad]).reshape(EROWS, 128)
    dste2d = jnp.concatenate([edge_index[1], pad]).reshape(EROWS, 128)
    return srce2d, dste2d


def kernel(x, edge_index, batch, W1, b1, Wlin, blin):
    srce2d, dste2d = _prep(edge_index)
    c_flat, cnt = _sc_coeffs(srce2d, dste2d, batch)
    return _tc_dense(c_flat, cnt, x, W1, b1, Wlin, blin)
